# Initial kernel scaffold; baseline (speedup 1.0000x reference)
import functools

import jax
import jax.numpy as jnp
from jax import lax
from jax.experimental import pallas as pl
from jax.experimental.pallas import tpu as pltpu
import jax.experimental.pallas.tpu_sc as plsc

N = 10000
E = 320000
D_IN = 128
H1 = 128
HID = 64

NUM_CORES = 2
NUM_SUBCORES = 16
NW = NUM_CORES * NUM_SUBCORES
EPW = E // NW
CHUNK = 80
NCHUNK = EPW // CHUNK
ROWS_PER_TILE = N // NUM_SUBCORES
COPY_ROWS = 125
NCOPY = ROWS_PER_TILE // COPY_ROWS

ROW_BLK = 1000
GRID = N // ROW_BLK


def _make_edge_pass(h):
  w = h + 16
  nsub = h // 16

  mesh = plsc.VectorSubcoreMesh(core_axis_name="c", subcore_axis_name="s")

  @functools.partial(
      pl.kernel,
      out_type=jax.ShapeDtypeStruct((2 * N, w), jnp.float32),
      mesh=mesh,
      scratch_types=[
          pltpu.VMEM((CHUNK,), jnp.int32),
          pltpu.VMEM((CHUNK,), jnp.int32),
          pltpu.VMEM((CHUNK,), jnp.float32),
          pltpu.VMEM((CHUNK, h), jnp.float32),
          pltpu.VMEM((CHUNK, h), jnp.float32),
          pltpu.VMEM((CHUNK, w), jnp.float32),
          pltpu.VMEM((h,), jnp.float32),
          pltpu.VMEM((h,), jnp.float32),
          pltpu.VMEM((COPY_ROWS, w), jnp.float32),
          pltpu.VMEM_SHARED((N, w), jnp.float32),
          pltpu.SemaphoreType.DMA,
          pltpu.SemaphoreType.DMA,
      ],
  )
  def edge_pass(xl_hbm, xr_hbm, src_hbm, dst_hbm, ea_hbm, we_hbm, att_hbm,
                num_hbm, src_v, dst_v, ea_v, xls_v, xrd_v, rows_v, we_v,
                att_v, bounce, num_sh, sem1, sem2):
    cid = lax.axis_index("c")
    sid = lax.axis_index("s")

    pltpu.sync_copy(we_hbm, we_v)
    pltpu.sync_copy(att_hbm, att_v)

    zvec = jnp.zeros((16,), jnp.float32)
    lane0 = (lax.iota(jnp.int32, 16) == 0).astype(jnp.float32)

    def zero_bounce(r, carry):
      for k in range(w // 16):
        bounce[r, pl.ds(k * 16, 16)] = zvec
      return carry

    lax.fori_loop(0, COPY_ROWS, zero_bounce, 0)

    def zero_spmem(j, carry):
      pltpu.sync_copy(
          bounce, num_sh.at[pl.ds(sid * ROWS_PER_TILE + j * COPY_ROWS,
                                  COPY_ROWS), :])
      return carry

    lax.fori_loop(0, NCOPY, zero_spmem, 0)
    plsc.subcore_barrier()

    wid = cid * NUM_SUBCORES + sid

    def edge_body(e, carry):
      ea_s = ea_v[e]
      acc = zvec
      for k in range(nsub):
        sl = pl.ds(k * 16, 16)
        v = xls_v[e, sl] + xrd_v[e, sl] + ea_s * we_v[sl]
        v = jnp.maximum(v, 0.2 * v)
        acc = acc + v * att_v[sl]
      ex = jnp.exp(jnp.full((16,), jnp.sum(acc), jnp.float32))
      for k in range(nsub):
        sl = pl.ds(k * 16, 16)
        rows_v[e, sl] = ex * xls_v[e, sl]
      rows_v[e, pl.ds(h, 16)] = ex * lane0
      return carry

    def chunk_body(c, carry):
      base = wid * EPW + c * CHUNK
      pltpu.sync_copy(src_hbm.at[pl.ds(base, CHUNK)], src_v)
      pltpu.sync_copy(dst_hbm.at[pl.ds(base, CHUNK)], dst_v)
      pltpu.sync_copy(ea_hbm.at[pl.ds(base, CHUNK)], ea_v)
      g1 = pltpu.async_copy(xl_hbm.at[src_v], xls_v, sem1)
      g2 = pltpu.async_copy(xr_hbm.at[dst_v], xrd_v, sem2)
      g1.wait()
      g2.wait()
      lax.fori_loop(0, CHUNK, edge_body, 0)
      pltpu.sync_copy(rows_v, num_sh.at[dst_v], add=True)
      return carry

    lax.fori_loop(0, NCHUNK, chunk_body, 0)
    plsc.subcore_barrier()

    def copy_out(j, carry):
      r0 = sid * ROWS_PER_TILE + j * COPY_ROWS
      pltpu.sync_copy(num_sh.at[pl.ds(r0, COPY_ROWS), :], bounce)
      pltpu.sync_copy(bounce, num_hbm.at[pl.ds(cid * N + r0, COPY_ROWS), :])
      return carry

    lax.fori_loop(0, NCOPY, copy_out, 0)

  return edge_pass


_edge_pass_1 = _make_edge_pass(H1)
_edge_pass_2 = _make_edge_pass(HID)


def _mean_body(w_ref, o_ref):
  o_ref[0, 0] = jnp.sum(w_ref[...]) / jnp.float32(E)


_mean_call = pl.pallas_call(
    _mean_body,
    out_shape=jax.ShapeDtypeStruct((1, 1), jnp.float32),
    out_specs=pl.BlockSpec(memory_space=pltpu.SMEM),
)


def _dense1_body(x_ref, wl_ref, bl_ref, wr_ref, br_ref, xl_ref, xr_ref):
  xb = x_ref[...]
  xl_ref[...] = jnp.dot(xb, wl_ref[...],
                        preferred_element_type=jnp.float32) + bl_ref[...]
  xr_ref[...] = jnp.dot(xb, wr_ref[...],
                        preferred_element_type=jnp.float32) + br_ref[...]


_dense1_call = pl.pallas_call(
    _dense1_body,
    grid=(GRID,),
    in_specs=[
        pl.BlockSpec((ROW_BLK, D_IN), lambda i: (i, 0)),
        pl.BlockSpec((D_IN, H1), lambda i: (0, 0)),
        pl.BlockSpec((1, H1), lambda i: (0, 0)),
        pl.BlockSpec((D_IN, H1), lambda i: (0, 0)),
        pl.BlockSpec((1, H1), lambda i: (0, 0)),
    ],
    out_specs=[
        pl.BlockSpec((ROW_BLK, H1), lambda i: (i, 0)),
        pl.BlockSpec((ROW_BLK, H1), lambda i: (i, 0)),
    ],
    out_shape=[
        jax.ShapeDtypeStruct((N, H1), jnp.float32),
        jax.ShapeDtypeStruct((N, H1), jnp.float32),
    ],
)


def _self_loop_ex(xl, xr, mean, we, att):
  v = xl + xr + mean * we
  v = jnp.maximum(v, 0.2 * v)
  return jnp.exp(jnp.dot(v, att, preferred_element_type=jnp.float32))


def _mid_body(p0_ref, p1_ref, xl_ref, xr_ref, mean_ref, we_ref, att_ref,
              b_ref, wl2_ref, bl2_ref, wr2_ref, br2_ref, xl2_ref, xr2_ref):
  xl = xl_ref[...]
  ex = _self_loop_ex(xl, xr_ref[...], mean_ref[0, 0], we_ref[...],
                     att_ref[...])
  num = p0_ref[:, pl.ds(0, H1)] + p1_ref[:, pl.ds(0, H1)] + ex * xl
  den = (p0_ref[:, pl.ds(H1, 1)] + p1_ref[:, pl.ds(H1, 1)] + ex
         + jnp.float32(1e-16))
  h = jnp.tanh(num / den + b_ref[...])
  xl2_ref[...] = jnp.dot(h, wl2_ref[...],
                         preferred_element_type=jnp.float32) + bl2_ref[...]
  xr2_ref[...] = jnp.dot(h, wr2_ref[...],
                         preferred_element_type=jnp.float32) + br2_ref[...]


_W1 = H1 + 16
_mid_call = pl.pallas_call(
    _mid_body,
    grid=(GRID,),
    in_specs=[
        pl.BlockSpec((ROW_BLK, _W1), lambda i: (i, 0)),
        pl.BlockSpec((ROW_BLK, _W1), lambda i: (i, 0)),
        pl.BlockSpec((ROW_BLK, H1), lambda i: (i, 0)),
        pl.BlockSpec((ROW_BLK, H1), lambda i: (i, 0)),
        pl.BlockSpec(memory_space=pltpu.SMEM),
        pl.BlockSpec((1, H1), lambda i: (0, 0)),
        pl.BlockSpec((H1, 1), lambda i: (0, 0)),
        pl.BlockSpec((1, H1), lambda i: (0, 0)),
        pl.BlockSpec((H1, HID), lambda i: (0, 0)),
        pl.BlockSpec((1, HID), lambda i: (0, 0)),
        pl.BlockSpec((H1, HID), lambda i: (0, 0)),
        pl.BlockSpec((1, HID), lambda i: (0, 0)),
    ],
    out_specs=[
        pl.BlockSpec((ROW_BLK, HID), lambda i: (i, 0)),
        pl.BlockSpec((ROW_BLK, HID), lambda i: (i, 0)),
    ],
    out_shape=[
        jax.ShapeDtypeStruct((N, HID), jnp.float32),
        jax.ShapeDtypeStruct((N, HID), jnp.float32),
    ],
)


def _fin_body(q0_ref, q1_ref, xl_ref, xr_ref, mean_ref, we_ref, att_ref,
              b_ref, o_ref):
  xl = xl_ref[...]
  ex = _self_loop_ex(xl, xr_ref[...], mean_ref[0, 0], we_ref[...],
                     att_ref[...])
  num = q0_ref[:, pl.ds(0, HID)] + q1_ref[:, pl.ds(0, HID)] + ex * xl
  den = (q0_ref[:, pl.ds(HID, 1)] + q1_ref[:, pl.ds(HID, 1)] + ex
         + jnp.float32(1e-16))
  o_ref[...] = num / den + b_ref[...]


_W2 = HID + 16
_fin_call = pl.pallas_call(
    _fin_body,
    grid=(GRID,),
    in_specs=[
        pl.BlockSpec((ROW_BLK, _W2), lambda i: (i, 0)),
        pl.BlockSpec((ROW_BLK, _W2), lambda i: (i, 0)),
        pl.BlockSpec((ROW_BLK, HID), lambda i: (i, 0)),
        pl.BlockSpec((ROW_BLK, HID), lambda i: (i, 0)),
        pl.BlockSpec(memory_space=pltpu.SMEM),
        pl.BlockSpec((1, HID), lambda i: (0, 0)),
        pl.BlockSpec((HID, 1), lambda i: (0, 0)),
        pl.BlockSpec((1, HID), lambda i: (0, 0)),
    ],
    out_specs=pl.BlockSpec((ROW_BLK, HID), lambda i: (i, 0)),
    out_shape=jax.ShapeDtypeStruct((N, HID), jnp.float32),
)


@jax.jit
def kernel(x, edge_idx, edge_w, Wl1, bl1, Wr1, br1, We1, att1, b1,
           Wl2, bl2, Wr2, br2, We2, att2, b2):
  src = edge_idx[0]
  dst = edge_idx[1]
  ea = edge_w[:, 0]

  mean = _mean_call(edge_w.reshape(E // D_IN, D_IN))
  xl1, xr1 = _dense1_call(x, Wl1, bl1.reshape(1, H1), Wr1,
                          br1.reshape(1, H1))

  num1 = _edge_pass_1(xl1, xr1, src, dst, ea, We1.reshape(H1), att1)
  xl2, xr2 = _mid_call(num1[:N], num1[N:], xl1, xr1, mean, We1,
                       att1.reshape(H1, 1), b1.reshape(1, H1),
                       Wl2, bl2.reshape(1, HID), Wr2, br2.reshape(1, HID))

  num2 = _edge_pass_2(xl2, xr2, src, dst, ea, We2.reshape(HID), att2)
  out = _fin_call(num2[:N], num2[N:], xl2, xr2, mean, We2,
                  att2.reshape(HID, 1), b2.reshape(1, HID))
  return out


# speedup vs baseline: 13.0520x; 13.0520x over previous
import functools

import jax
import jax.numpy as jnp
from jax import lax
from jax.experimental import pallas as pl
from jax.experimental.pallas import tpu as pltpu
import jax.experimental.pallas.tpu_sc as plsc

N = 10000
E = 320000
D_IN = 128
H1 = 128
HID = 64

NUM_CORES = 2
NUM_SUBCORES = 16
NW = NUM_CORES * NUM_SUBCORES
EPW = E // NW
CHUNK = 80
NCHUNK = EPW // CHUNK
ROWS_PER_TILE = N // NUM_SUBCORES
COPY_ROWS = 25
NCOPY = ROWS_PER_TILE // COPY_ROWS

ROW_BLK = 1000
GRID = N // ROW_BLK


def _make_edge_pass(h):
  nsub = h // 16
  ngrp = CHUNK // 16

  mesh = plsc.VectorSubcoreMesh(core_axis_name="c", subcore_axis_name="s")

  @functools.partial(
      pl.kernel,
      out_type=(jax.ShapeDtypeStruct((2 * N, h), jnp.float32),
                jax.ShapeDtypeStruct((NW, N), jnp.float32)),
      mesh=mesh,
      compiler_params=pltpu.CompilerParams(use_tc_tiling_on_sc=False,
                                           needs_layout_passes=False),
      scratch_types=[
          pltpu.VMEM((CHUNK,), jnp.int32),
          pltpu.VMEM((CHUNK,), jnp.int32),
          pltpu.VMEM((CHUNK + 16,), jnp.float32),
          pltpu.VMEM((CHUNK, h), jnp.float32),
          pltpu.VMEM((CHUNK, h), jnp.float32),
          pltpu.VMEM((CHUNK, h), jnp.float32),
          pltpu.VMEM((h,), jnp.float32),
          pltpu.VMEM((h,), jnp.float32),
          pltpu.VMEM((COPY_ROWS, h), jnp.float32),
          pltpu.VMEM((N,), jnp.float32),
          pltpu.VMEM_SHARED((N, h), jnp.float32),
          pltpu.SemaphoreType.DMA,
          pltpu.SemaphoreType.DMA,
      ],
  )
  def edge_pass(xl_hbm, xr_hbm, src_hbm, dst_hbm, ea_hbm, we_hbm, att_hbm,
                num_hbm, den_hbm, src_v, dst_v, ea_v, xls_v, xrd_v, rows_v,
                we_v, att_v, bounce, den_local, num_sh, sem1, sem2):
    cid = lax.axis_index("c")
    sid = lax.axis_index("s")

    pltpu.sync_copy(we_hbm, we_v)
    pltpu.sync_copy(att_hbm, att_v)

    zvec = jnp.zeros((16,), jnp.float32)
    lane = lax.iota(jnp.int32, 16)

    def zero_bounce(r, carry):
      for k in range(nsub):
        bounce[r, pl.ds(k * 16, 16)] = zvec
      return carry

    lax.fori_loop(0, COPY_ROWS, zero_bounce, 0)

    def zero_den(r, carry):
      den_local[pl.ds(r * 16, 16)] = zvec
      return carry

    lax.fori_loop(0, N // 16, zero_den, 0)

    def zero_spmem(j, carry):
      pltpu.sync_copy(
          bounce, num_sh.at[pl.ds(sid * ROWS_PER_TILE + j * COPY_ROWS,
                                  COPY_ROWS), :])
      return carry

    lax.fori_loop(0, NCOPY, zero_spmem, 0)
    plsc.subcore_barrier()

    wid = cid * NUM_SUBCORES + sid

    def group_body(g, carry):
      e0 = g * 16
      logits = zvec
      for j in range(16):
        e = e0 + j
        ea_s = ea_v[pl.ds(e, 16)][0]
        acc = zvec
        for k in range(nsub):
          sl = pl.ds(k * 16, 16)
          v = xls_v[e, sl] + xrd_v[e, sl] + ea_s * we_v[sl]
          v = jnp.maximum(v, 0.2 * v)
          acc = acc + v * att_v[sl]
        s = jnp.full((16,), jnp.sum(acc), jnp.float32)
        logits = jnp.where(lane == j, s, logits)
      exg = jnp.exp(logits)
      dst16 = dst_v[pl.ds(e0, 16)]
      for j in range(16):
        plsc.addupdate_scatter(den_local, [dst16], exg, mask=lane == j)
      for j in range(16):
        e = e0 + j
        exj = exg[j]
        for k in range(nsub):
          sl = pl.ds(k * 16, 16)
          rows_v[e, sl] = exj * xls_v[e, sl]
      return carry

    def chunk_body(c, carry):
      base = wid * EPW + c * CHUNK
      pltpu.sync_copy(src_hbm.at[pl.ds(base, CHUNK)], src_v)
      pltpu.sync_copy(dst_hbm.at[pl.ds(base, CHUNK)], dst_v)
      pltpu.sync_copy(ea_hbm.at[pl.ds(base, CHUNK)],
                      ea_v.at[pl.ds(0, CHUNK)])
      g1 = pltpu.async_copy(xl_hbm.at[src_v], xls_v, sem1)
      g2 = pltpu.async_copy(xr_hbm.at[dst_v], xrd_v, sem2)
      g1.wait()
      g2.wait()
      lax.fori_loop(0, ngrp, group_body, 0)
      pltpu.sync_copy(rows_v, num_sh.at[dst_v], add=True)
      return carry

    lax.fori_loop(0, NCHUNK, chunk_body, 0)
    plsc.subcore_barrier()

    pltpu.sync_copy(den_local, den_hbm.at[wid])

    def copy_out(j, carry):
      r0 = sid * ROWS_PER_TILE + j * COPY_ROWS
      pltpu.sync_copy(num_sh.at[pl.ds(r0, COPY_ROWS), :], bounce)
      pltpu.sync_copy(bounce, num_hbm.at[pl.ds(cid * N + r0, COPY_ROWS), :])
      return carry

    lax.fori_loop(0, NCOPY, copy_out, 0)

  return edge_pass


_edge_pass_1 = _make_edge_pass(H1)
_edge_pass_2 = _make_edge_pass(HID)


def _mean_body(w_ref, o_ref):
  o_ref[0, 0] = jnp.sum(w_ref[...]) / jnp.float32(E)


_mean_call = pl.pallas_call(
    _mean_body,
    out_shape=jax.ShapeDtypeStruct((1, 1), jnp.float32),
    out_specs=pl.BlockSpec(memory_space=pltpu.SMEM),
)


def _dense1_body(x_ref, wl_ref, bl_ref, wr_ref, br_ref, xl_ref, xr_ref):
  xb = x_ref[...]
  xl_ref[...] = jnp.dot(xb, wl_ref[...],
                        preferred_element_type=jnp.float32) + bl_ref[...]
  xr_ref[...] = jnp.dot(xb, wr_ref[...],
                        preferred_element_type=jnp.float32) + br_ref[...]


_dense1_call = pl.pallas_call(
    _dense1_body,
    grid=(GRID,),
    in_specs=[
        pl.BlockSpec((ROW_BLK, D_IN), lambda i: (i, 0)),
        pl.BlockSpec((D_IN, H1), lambda i: (0, 0)),
        pl.BlockSpec((1, H1), lambda i: (0, 0)),
        pl.BlockSpec((D_IN, H1), lambda i: (0, 0)),
        pl.BlockSpec((1, H1), lambda i: (0, 0)),
    ],
    out_specs=[
        pl.BlockSpec((ROW_BLK, H1), lambda i: (i, 0)),
        pl.BlockSpec((ROW_BLK, H1), lambda i: (i, 0)),
    ],
    out_shape=[
        jax.ShapeDtypeStruct((N, H1), jnp.float32),
        jax.ShapeDtypeStruct((N, H1), jnp.float32),
    ],
)


def _self_loop_ex(xl, xr, mean, we, att):
  v = xl + xr + mean * we
  v = jnp.maximum(v, 0.2 * v)
  return jnp.exp(jnp.dot(v, att, preferred_element_type=jnp.float32))


def _mid_body(p0_ref, p1_ref, dp_ref, xl_ref, xr_ref, mean_ref, we_ref,
              att_ref, b_ref, wl2_ref, bl2_ref, wr2_ref, br2_ref, xl2_ref,
              xr2_ref):
  xl = xl_ref[...]
  ex = _self_loop_ex(xl, xr_ref[...], mean_ref[0, 0], we_ref[...],
                     att_ref[...])
  num = p0_ref[...] + p1_ref[...] + ex * xl
  den = (jnp.sum(dp_ref[...], axis=1, keepdims=True) + ex
         + jnp.float32(1e-16))
  h = jnp.tanh(num / den + b_ref[...])
  xl2_ref[...] = jnp.dot(h, wl2_ref[...],
                         preferred_element_type=jnp.float32) + bl2_ref[...]
  xr2_ref[...] = jnp.dot(h, wr2_ref[...],
                         preferred_element_type=jnp.float32) + br2_ref[...]


_mid_call = pl.pallas_call(
    _mid_body,
    grid=(GRID,),
    in_specs=[
        pl.BlockSpec((ROW_BLK, H1), lambda i: (i, 0)),
        pl.BlockSpec((ROW_BLK, H1), lambda i: (i + GRID, 0)),
        pl.BlockSpec((ROW_BLK, NW), lambda i: (i, 0)),
        pl.BlockSpec((ROW_BLK, H1), lambda i: (i, 0)),
        pl.BlockSpec((ROW_BLK, H1), lambda i: (i, 0)),
        pl.BlockSpec(memory_space=pltpu.SMEM),
        pl.BlockSpec((1, H1), lambda i: (0, 0)),
        pl.BlockSpec((H1, 1), lambda i: (0, 0)),
        pl.BlockSpec((1, H1), lambda i: (0, 0)),
        pl.BlockSpec((H1, HID), lambda i: (0, 0)),
        pl.BlockSpec((1, HID), lambda i: (0, 0)),
        pl.BlockSpec((H1, HID), lambda i: (0, 0)),
        pl.BlockSpec((1, HID), lambda i: (0, 0)),
    ],
    out_specs=[
        pl.BlockSpec((ROW_BLK, HID), lambda i: (i, 0)),
        pl.BlockSpec((ROW_BLK, HID), lambda i: (i, 0)),
    ],
    out_shape=[
        jax.ShapeDtypeStruct((N, HID), jnp.float32),
        jax.ShapeDtypeStruct((N, HID), jnp.float32),
    ],
)


def _fin_body(q0_ref, q1_ref, dp_ref, xl_ref, xr_ref, mean_ref, we_ref,
              att_ref, b_ref, o_ref):
  xl = xl_ref[...]
  ex = _self_loop_ex(xl, xr_ref[...], mean_ref[0, 0], we_ref[...],
                     att_ref[...])
  num = q0_ref[...] + q1_ref[...] + ex * xl
  den = (jnp.sum(dp_ref[...], axis=1, keepdims=True) + ex
         + jnp.float32(1e-16))
  o_ref[...] = num / den + b_ref[...]


_fin_call = pl.pallas_call(
    _fin_body,
    grid=(GRID,),
    in_specs=[
        pl.BlockSpec((ROW_BLK, HID), lambda i: (i, 0)),
        pl.BlockSpec((ROW_BLK, HID), lambda i: (i + GRID, 0)),
        pl.BlockSpec((ROW_BLK, NW), lambda i: (i, 0)),
        pl.BlockSpec((ROW_BLK, HID), lambda i: (i, 0)),
        pl.BlockSpec((ROW_BLK, HID), lambda i: (i, 0)),
        pl.BlockSpec(memory_space=pltpu.SMEM),
        pl.BlockSpec((1, HID), lambda i: (0, 0)),
        pl.BlockSpec((HID, 1), lambda i: (0, 0)),
        pl.BlockSpec((1, HID), lambda i: (0, 0)),
    ],
    out_specs=pl.BlockSpec((ROW_BLK, HID), lambda i: (i, 0)),
    out_shape=jax.ShapeDtypeStruct((N, HID), jnp.float32),
)


@jax.jit
def kernel(x, edge_idx, edge_w, Wl1, bl1, Wr1, br1, We1, att1, b1,
           Wl2, bl2, Wr2, br2, We2, att2, b2):
  src = edge_idx[0]
  dst = edge_idx[1]
  ea = edge_w[:, 0]

  mean = _mean_call(edge_w.reshape(E // D_IN, D_IN))
  xl1, xr1 = _dense1_call(x, Wl1, bl1.reshape(1, H1), Wr1,
                          br1.reshape(1, H1))

  num1, den1 = _edge_pass_1(xl1, xr1, src, dst, ea, We1.reshape(H1), att1)
  xl2, xr2 = _mid_call(num1, num1, den1.T, xl1, xr1, mean, We1,
                       att1.reshape(H1, 1), b1.reshape(1, H1),
                       Wl2, bl2.reshape(1, HID), Wr2, br2.reshape(1, HID))

  num2, den2 = _edge_pass_2(xl2, xr2, src, dst, ea, We2.reshape(HID), att2)
  out = _fin_call(num2, num2, den2.T, xl2, xr2, mean, We2,
                  att2.reshape(HID, 1), b2.reshape(1, HID))
  return out


# speedup vs baseline: 13.1951x; 1.0110x over previous
import functools

import jax
import jax.numpy as jnp
from jax import lax
from jax.experimental import pallas as pl
from jax.experimental.pallas import tpu as pltpu
import jax.experimental.pallas.tpu_sc as plsc

N = 10000
E = 320000
D_IN = 128
H1 = 128
HID = 64

NUM_CORES = 2
NUM_SUBCORES = 16
NW = NUM_CORES * NUM_SUBCORES
EPW = E // NW
CHUNK = 80
NCHUNK = EPW // CHUNK
ROWS_PER_TILE = N // NUM_SUBCORES
COPY_ROWS = 25
NCOPY = ROWS_PER_TILE // COPY_ROWS

ROW_BLK = 1000
GRID = N // ROW_BLK


def _make_edge_pass(h):
  nsub = h // 16
  ngrp = CHUNK // 16

  mesh = plsc.VectorSubcoreMesh(core_axis_name="c", subcore_axis_name="s")

  @functools.partial(
      pl.kernel,
      out_type=(jax.ShapeDtypeStruct((2 * N, h), jnp.float32),
                jax.ShapeDtypeStruct((NW, N), jnp.float32)),
      mesh=mesh,
      compiler_params=pltpu.CompilerParams(use_tc_tiling_on_sc=False,
                                           needs_layout_passes=False),
      scratch_types=[
          [pltpu.VMEM((CHUNK,), jnp.int32)] * 2,
          [pltpu.VMEM((CHUNK,), jnp.int32)] * 2,
          [pltpu.VMEM((CHUNK,), jnp.float32)] * 2,
          [pltpu.VMEM((CHUNK, h), jnp.float32)] * 2,
          pltpu.VMEM((CHUNK, h), jnp.float32),
          pltpu.VMEM((h,), jnp.float32),
          pltpu.VMEM((h,), jnp.float32),
          pltpu.VMEM((COPY_ROWS, h), jnp.float32),
          pltpu.VMEM((N,), jnp.float32),
          pltpu.VMEM_SHARED((N, h), jnp.float32),
          [pltpu.SemaphoreType.DMA] * 2,
          pltpu.SemaphoreType.DMA,
      ],
  )
  def edge_pass(xl_hbm, xr_hbm, src_hbm, dst_hbm, ea_hbm, we_hbm, att_hbm,
                num_hbm, den_hbm, src_b, dst_b, ea_b, xls_b, xrd_v,
                we_v, att_v, bounce, den_local, num_sh, gx_s, gr_sem):
    cid = lax.axis_index("c")
    sid = lax.axis_index("s")

    pltpu.sync_copy(we_hbm, we_v)
    pltpu.sync_copy(att_hbm, att_v)

    zvec = jnp.zeros(16, jnp.float32)
    lane = lax.iota(jnp.int32, 16)
    wvs = [we_v[pl.ds(k * 16, 16)] for k in range(nsub)]
    avs = [att_v[pl.ds(k * 16, 16)] for k in range(nsub)]

    def zero_bounce(r, carry):
      for k in range(nsub):
        bounce[r, pl.ds(k * 16, 16)] = zvec
      return carry

    lax.fori_loop(0, COPY_ROWS, zero_bounce, 0)

    def zero_den(r, carry):
      den_local[pl.ds(r * 16, 16)] = zvec
      return carry

    lax.fori_loop(0, N // 16, zero_den, 0)

    def zero_spmem(j, carry):
      pltpu.sync_copy(
          bounce, num_sh.at[pl.ds(sid * ROWS_PER_TILE + j * COPY_ROWS,
                                  COPY_ROWS), :])
      return carry

    lax.fori_loop(0, NCOPY, zero_spmem, 0)
    plsc.subcore_barrier()

    wid = cid * NUM_SUBCORES + sid

    def stage_a(c, b):
      base = wid * EPW + c * CHUNK
      pltpu.sync_copy(src_hbm.at[pl.ds(base, CHUNK)], src_b[b])
      pltpu.sync_copy(dst_hbm.at[pl.ds(base, CHUNK)], dst_b[b])
      pltpu.sync_copy(ea_hbm.at[pl.ds(base, CHUNK)], ea_b[b])
      pltpu.async_copy(xl_hbm.at[src_b[b]], xls_b[b], gx_s[b])

    def stage_b(b):
      pltpu.async_copy(xr_hbm.at[dst_b[b]], xrd_v, gr_sem).wait()
      pltpu.make_async_copy(xl_hbm.at[src_b[b]], xls_b[b], gx_s[b]).wait()
      xls_v = xls_b[b]

      def group_body(g, carry):
        e0 = g * 16
        ea16 = ea_b[b][pl.ds(e0, 16)]
        dst16 = dst_b[b][pl.ds(e0, 16)]
        logits = zvec
        for j in range(16):
          e = e0 + j
          ea_s = ea16[j]
          acc = zvec
          for k in range(nsub):
            sl = pl.ds(k * 16, 16)
            v = xls_v[e, sl] + xrd_v[e, sl] + ea_s * wvs[k]
            v = jnp.maximum(v, 0.2 * v)
            acc = acc + v * avs[k]
          s = jnp.full((16,), jnp.sum(acc), jnp.float32)
          logits = jnp.where(lane == j, s, logits)
        exg = jnp.exp(logits)
        for j in range(16):
          plsc.addupdate_scatter(den_local, [dst16], exg, mask=lane == j)
        for j in range(16):
          e = e0 + j
          exj = exg[j]
          for k in range(nsub):
            sl = pl.ds(k * 16, 16)
            xls_v[e, sl] = exj * xls_v[e, sl]
        return carry

      lax.fori_loop(0, ngrp, group_body, 0)
      pltpu.sync_copy(xls_v, num_sh.at[dst_b[b]], add=True)

    stage_a(0, 0)

    def pipe_body(cc, carry):
      c0 = 2 * cc
      stage_a(c0 + 1, 1)
      stage_b(0)
      stage_a(c0 + 2, 0)
      stage_b(1)
      return carry

    lax.fori_loop(0, (NCHUNK - 1) // 2, pipe_body, 0)
    stage_b(0)
    if NCHUNK % 2 == 0:
      stage_a(NCHUNK - 1, 1)
      stage_b(1)
    plsc.subcore_barrier()

    pltpu.sync_copy(den_local, den_hbm.at[wid])

    def copy_out(j, carry):
      r0 = sid * ROWS_PER_TILE + j * COPY_ROWS
      pltpu.sync_copy(num_sh.at[pl.ds(r0, COPY_ROWS), :], bounce)
      pltpu.sync_copy(bounce, num_hbm.at[pl.ds(cid * N + r0, COPY_ROWS), :])
      return carry

    lax.fori_loop(0, NCOPY, copy_out, 0)

  return edge_pass


_edge_pass_1 = _make_edge_pass(H1)
_edge_pass_2 = _make_edge_pass(HID)


def _mean_body(w_ref, o_ref):
  o_ref[0, 0] = jnp.sum(w_ref[...]) / jnp.float32(E)


_mean_call = pl.pallas_call(
    _mean_body,
    out_shape=jax.ShapeDtypeStruct((1, 1), jnp.float32),
    out_specs=pl.BlockSpec(memory_space=pltpu.SMEM),
)


def _dense1_body(x_ref, wl_ref, bl_ref, wr_ref, br_ref, xl_ref, xr_ref):
  xb = x_ref[...]
  xl_ref[...] = jnp.dot(xb, wl_ref[...],
                        preferred_element_type=jnp.float32) + bl_ref[...]
  xr_ref[...] = jnp.dot(xb, wr_ref[...],
                        preferred_element_type=jnp.float32) + br_ref[...]


_dense1_call = pl.pallas_call(
    _dense1_body,
    grid=(GRID,),
    in_specs=[
        pl.BlockSpec((ROW_BLK, D_IN), lambda i: (i, 0)),
        pl.BlockSpec((D_IN, H1), lambda i: (0, 0)),
        pl.BlockSpec((1, H1), lambda i: (0, 0)),
        pl.BlockSpec((D_IN, H1), lambda i: (0, 0)),
        pl.BlockSpec((1, H1), lambda i: (0, 0)),
    ],
    out_specs=[
        pl.BlockSpec((ROW_BLK, H1), lambda i: (i, 0)),
        pl.BlockSpec((ROW_BLK, H1), lambda i: (i, 0)),
    ],
    out_shape=[
        jax.ShapeDtypeStruct((N, H1), jnp.float32),
        jax.ShapeDtypeStruct((N, H1), jnp.float32),
    ],
)


def _self_loop_ex(xl, xr, mean, we, att):
  v = xl + xr + mean * we
  v = jnp.maximum(v, 0.2 * v)
  return jnp.exp(jnp.dot(v, att, preferred_element_type=jnp.float32))


def _mid_body(p0_ref, p1_ref, dp_ref, xl_ref, xr_ref, mean_ref, we_ref,
              att_ref, b_ref, wl2_ref, bl2_ref, wr2_ref, br2_ref, xl2_ref,
              xr2_ref):
  xl = xl_ref[...]
  ex = _self_loop_ex(xl, xr_ref[...], mean_ref[0, 0], we_ref[...],
                     att_ref[...])
  num = p0_ref[...] + p1_ref[...] + ex * xl
  den = (jnp.sum(dp_ref[...], axis=1, keepdims=True) + ex
         + jnp.float32(1e-16))
  h = jnp.tanh(num / den + b_ref[...])
  xl2_ref[...] = jnp.dot(h, wl2_ref[...],
                         preferred_element_type=jnp.float32) + bl2_ref[...]
  xr2_ref[...] = jnp.dot(h, wr2_ref[...],
                         preferred_element_type=jnp.float32) + br2_ref[...]


_mid_call = pl.pallas_call(
    _mid_body,
    grid=(GRID,),
    in_specs=[
        pl.BlockSpec((ROW_BLK, H1), lambda i: (i, 0)),
        pl.BlockSpec((ROW_BLK, H1), lambda i: (i + GRID, 0)),
        pl.BlockSpec((ROW_BLK, NW), lambda i: (i, 0)),
        pl.BlockSpec((ROW_BLK, H1), lambda i: (i, 0)),
        pl.BlockSpec((ROW_BLK, H1), lambda i: (i, 0)),
        pl.BlockSpec(memory_space=pltpu.SMEM),
        pl.BlockSpec((1, H1), lambda i: (0, 0)),
        pl.BlockSpec((H1, 1), lambda i: (0, 0)),
        pl.BlockSpec((1, H1), lambda i: (0, 0)),
        pl.BlockSpec((H1, HID), lambda i: (0, 0)),
        pl.BlockSpec((1, HID), lambda i: (0, 0)),
        pl.BlockSpec((H1, HID), lambda i: (0, 0)),
        pl.BlockSpec((1, HID), lambda i: (0, 0)),
    ],
    out_specs=[
        pl.BlockSpec((ROW_BLK, HID), lambda i: (i, 0)),
        pl.BlockSpec((ROW_BLK, HID), lambda i: (i, 0)),
    ],
    out_shape=[
        jax.ShapeDtypeStruct((N, HID), jnp.float32),
        jax.ShapeDtypeStruct((N, HID), jnp.float32),
    ],
)


def _fin_body(q0_ref, q1_ref, dp_ref, xl_ref, xr_ref, mean_ref, we_ref,
              att_ref, b_ref, o_ref):
  xl = xl_ref[...]
  ex = _self_loop_ex(xl, xr_ref[...], mean_ref[0, 0], we_ref[...],
                     att_ref[...])
  num = q0_ref[...] + q1_ref[...] + ex * xl
  den = (jnp.sum(dp_ref[...], axis=1, keepdims=True) + ex
         + jnp.float32(1e-16))
  o_ref[...] = num / den + b_ref[...]


_fin_call = pl.pallas_call(
    _fin_body,
    grid=(GRID,),
    in_specs=[
        pl.BlockSpec((ROW_BLK, HID), lambda i: (i, 0)),
        pl.BlockSpec((ROW_BLK, HID), lambda i: (i + GRID, 0)),
        pl.BlockSpec((ROW_BLK, NW), lambda i: (i, 0)),
        pl.BlockSpec((ROW_BLK, HID), lambda i: (i, 0)),
        pl.BlockSpec((ROW_BLK, HID), lambda i: (i, 0)),
        pl.BlockSpec(memory_space=pltpu.SMEM),
        pl.BlockSpec((1, HID), lambda i: (0, 0)),
        pl.BlockSpec((HID, 1), lambda i: (0, 0)),
        pl.BlockSpec((1, HID), lambda i: (0, 0)),
    ],
    out_specs=pl.BlockSpec((ROW_BLK, HID), lambda i: (i, 0)),
    out_shape=jax.ShapeDtypeStruct((N, HID), jnp.float32),
)


@jax.jit
def kernel(x, edge_idx, edge_w, Wl1, bl1, Wr1, br1, We1, att1, b1,
           Wl2, bl2, Wr2, br2, We2, att2, b2):
  src = edge_idx[0]
  dst = edge_idx[1]
  ea = edge_w[:, 0]

  mean = _mean_call(edge_w.reshape(E // D_IN, D_IN))
  xl1, xr1 = _dense1_call(x, Wl1, bl1.reshape(1, H1), Wr1,
                          br1.reshape(1, H1))

  num1, den1 = _edge_pass_1(xl1, xr1, src, dst, ea, We1.reshape(H1), att1)
  xl2, xr2 = _mid_call(num1, num1, den1.T, xl1, xr1, mean, We1,
                       att1.reshape(H1, 1), b1.reshape(1, H1),
                       Wl2, bl2.reshape(1, HID), Wr2, br2.reshape(1, HID))

  num2, den2 = _edge_pass_2(xl2, xr2, src, dst, ea, We2.reshape(HID), att2)
  out = _fin_call(num2, num2, den2.T, xl2, xr2, mean, We2,
                  att2.reshape(HID, 1), b2.reshape(1, HID))
  return out
